# baseline (speedup 1.0000x reference)
import functools

import jax
import jax.numpy as jnp
from jax import lax
from jax.experimental import pallas as pl
from jax.experimental.pallas import tpu as pltpu


def _rb_kernel(x_ref, w1_ref, w2_ref, out_ref, h_ref, *, H, W, B):
    HW = H * W
    C = x_ref.shape[1]

    idx = lax.broadcasted_iota(jnp.int32, (1, HW), 1)
    row = idx // W
    colv = idx % W
    one = jnp.full((1, HW), 1.0, jnp.float32)
    zero = jnp.zeros((1, HW), jnp.float32)

    taps = []
    for kx in range(3):
        dx = kx - 1
        for ky in range(3):
            dy = ky - 1
            off = dy * W + dx
            if off == 0:
                taps.append((0, None))
                continue
            ok = (row + dy >= 0) & (row + dy < H) & \
                 (colv + dx >= 0) & (colv + dx < W)
            taps.append((off, jnp.where(ok, one, zero).astype(jnp.bfloat16)))

    w1 = w1_ref[...]
    w2 = w2_ref[...]

    def tap(v, t):
        off, mask = taps[t]
        if off == 0:
            return v
        k = off % HW
        s = jnp.concatenate([v[:, k:], v[:, :k]], axis=1)
        return s * mask

    def conv(v, wk):
        p = [tap(v, t) for t in range(9)]
        acc = jnp.dot(wk[:, 0 * C:2 * C],
                      jnp.concatenate([p[0], p[1]], axis=0),
                      preferred_element_type=jnp.float32)
        acc = acc + jnp.dot(wk[:, 2 * C:4 * C],
                            jnp.concatenate([p[2], p[3]], axis=0),
                            preferred_element_type=jnp.float32)
        acc = acc + jnp.dot(wk[:, 4 * C:6 * C],
                            jnp.concatenate([p[4], p[5]], axis=0),
                            preferred_element_type=jnp.float32)
        acc = acc + jnp.dot(wk[:, 6 * C:8 * C],
                            jnp.concatenate([p[6], p[7]], axis=0),
                            preferred_element_type=jnp.float32)
        acc = acc + jnp.dot(wk[:, 8 * C:9 * C], p[8],
                            preferred_element_type=jnp.float32)
        return acc

    for b in range(B):
        h = conv(x_ref[b].astype(jnp.bfloat16), w1)
        h_ref[b] = jnp.maximum(h, 0.0).astype(jnp.bfloat16)
    for b in range(B):
        y = conv(h_ref[b], w2)
        out_ref[b] = x_ref[b] + y


def _batch_block(N):
    for d in (16, 8, 4, 2, 1):
        if N % d == 0 and (N // d >= 2 or N < 2):
            return d
    return 1


@jax.jit
def _resblock(x_nchw, w1_oihw, w2_oihw):
    N, C, H, W = x_nchw.shape
    HW = H * W
    B = _batch_block(N)
    G = N // B

    x_flat = x_nchw.reshape(N, C, HW)
    w1m = jnp.transpose(w1_oihw, (0, 3, 2, 1)).reshape(C, 9 * C)
    w2m = jnp.transpose(w2_oihw, (0, 3, 2, 1)).reshape(C, 9 * C)
    w1m = w1m.astype(jnp.bfloat16)
    w2m = w2m.astype(jnp.bfloat16)

    body = functools.partial(_rb_kernel, H=H, W=W, B=B)

    out_flat = pl.pallas_call(
        body,
        out_shape=jax.ShapeDtypeStruct((N, C, HW), x_nchw.dtype),
        grid_spec=pltpu.PrefetchScalarGridSpec(
            num_scalar_prefetch=0,
            grid=(G,),
            in_specs=[
                pl.BlockSpec((B, C, HW), lambda g: (g, 0, 0)),
                pl.BlockSpec((C, 9 * C), lambda g: (0, 0)),
                pl.BlockSpec((C, 9 * C), lambda g: (0, 0)),
            ],
            out_specs=pl.BlockSpec((B, C, HW), lambda g: (g, 0, 0)),
            scratch_shapes=[
                pltpu.VMEM((B, C, HW), jnp.bfloat16),
            ],
        ),
        compiler_params=pltpu.CompilerParams(
            dimension_semantics=("parallel",),
            vmem_limit_bytes=48 * 1024 * 1024,
        ),
    )(x_flat, w1m, w2m)

    return out_flat.reshape(N, C, H, W)


def kernel(x_nchw, w1_oihw, w2_oihw):
    return _resblock(x_nchw, w1_oihw, w2_oihw)


# speedup vs baseline: 1.0108x; 1.0108x over previous
import functools

import jax
import jax.numpy as jnp
from jax import lax
from jax.experimental import pallas as pl
from jax.experimental.pallas import tpu as pltpu


def _rb_kernel(x_ref, w1_ref, w2_ref, out_ref, h_ref, *, H, W, B):
    HW = H * W
    C = x_ref.shape[1]

    idx = lax.broadcasted_iota(jnp.int32, (1, HW), 1)
    row = idx // W
    colv = idx % W
    one = jnp.full((1, HW), 1.0, jnp.float32)
    zero = jnp.zeros((1, HW), jnp.float32)

    taps = []
    for kx in range(3):
        dx = kx - 1
        for ky in range(3):
            dy = ky - 1
            off = dy * W + dx
            if off == 0:
                taps.append((0, None))
                continue
            ok = (row + dy >= 0) & (row + dy < H) & \
                 (colv + dx >= 0) & (colv + dx < W)
            taps.append((off, jnp.where(ok, one, zero).astype(jnp.bfloat16)))

    w1 = w1_ref[...]
    w2 = w2_ref[...]

    def tap(v, t):
        off, mask = taps[t]
        if off == 0:
            return v
        k = off % HW
        s = jnp.concatenate([v[:, k:], v[:, :k]], axis=1)
        return s * mask

    def conv(v, wk):
        p = [tap(v, t) for t in range(9)]
        acc = jnp.dot(wk[:, 0 * C:2 * C],
                      jnp.concatenate([p[0], p[1]], axis=0),
                      preferred_element_type=jnp.float32)
        acc = acc + jnp.dot(wk[:, 2 * C:4 * C],
                            jnp.concatenate([p[2], p[3]], axis=0),
                            preferred_element_type=jnp.float32)
        acc = acc + jnp.dot(wk[:, 4 * C:6 * C],
                            jnp.concatenate([p[4], p[5]], axis=0),
                            preferred_element_type=jnp.float32)
        acc = acc + jnp.dot(wk[:, 6 * C:8 * C],
                            jnp.concatenate([p[6], p[7]], axis=0),
                            preferred_element_type=jnp.float32)
        acc = acc + jnp.dot(wk[:, 8 * C:9 * C], p[8],
                            preferred_element_type=jnp.float32)
        return acc

    for b in range(B):
        h = conv(x_ref[b].astype(jnp.bfloat16), w1)
        h_ref[b] = jnp.maximum(h, 0.0).astype(jnp.bfloat16)
    for b in range(B):
        y = conv(h_ref[b], w2)
        out_ref[b] = x_ref[b] + y


def _batch_block(N):
    for d in (8, 4, 2, 1):
        if N % d == 0 and (N // d >= 2 or N < 2):
            return d
    return 1


@jax.jit
def _resblock(x_nchw, w1_oihw, w2_oihw):
    N, C, H, W = x_nchw.shape
    HW = H * W
    B = _batch_block(N)
    G = N // B

    x_flat = x_nchw.reshape(N, C, HW)
    w1m = jnp.transpose(w1_oihw, (0, 3, 2, 1)).reshape(C, 9 * C)
    w2m = jnp.transpose(w2_oihw, (0, 3, 2, 1)).reshape(C, 9 * C)
    w1m = w1m.astype(jnp.bfloat16)
    w2m = w2m.astype(jnp.bfloat16)

    body = functools.partial(_rb_kernel, H=H, W=W, B=B)

    out_flat = pl.pallas_call(
        body,
        out_shape=jax.ShapeDtypeStruct((N, C, HW), x_nchw.dtype),
        grid_spec=pltpu.PrefetchScalarGridSpec(
            num_scalar_prefetch=0,
            grid=(G,),
            in_specs=[
                pl.BlockSpec((B, C, HW), lambda g: (g, 0, 0)),
                pl.BlockSpec((C, 9 * C), lambda g: (0, 0)),
                pl.BlockSpec((C, 9 * C), lambda g: (0, 0)),
            ],
            out_specs=pl.BlockSpec((B, C, HW), lambda g: (g, 0, 0)),
            scratch_shapes=[
                pltpu.VMEM((B, C, HW), jnp.bfloat16),
            ],
        ),
        compiler_params=pltpu.CompilerParams(
            dimension_semantics=("parallel",),
            vmem_limit_bytes=48 * 1024 * 1024,
        ),
    )(x_flat, w1m, w2m)

    return out_flat.reshape(N, C, H, W)


def kernel(x_nchw, w1_oihw, w2_oihw):
    return _resblock(x_nchw, w1_oihw, w2_oihw)
